# baseline (speedup 1.0000x reference)
import functools

import jax
import jax.numpy as jnp
from jax import lax
from jax.experimental import pallas as pl
from jax.experimental.pallas import tpu as pltpu
from jax.experimental.pallas import tpu_sc as plsc

NUM_COUPLES = 100000
EMBED_DIM = 128
BATCH = 16384

_IDX_CHUNK = 128


@functools.lru_cache(maxsize=None)
def _make_gather(batch: int, dim: int):
    info = plsc.get_sparse_core_info()
    num_workers = info.num_cores * info.num_subcores
    b_per_w = batch // num_workers
    assert batch % (8 * num_workers) == 0
    n_chunks = b_per_w // _IDX_CHUNK
    assert b_per_w % _IDX_CHUNK == 0
    mesh = plsc.VectorSubcoreMesh(core_axis_name="c", subcore_axis_name="s")

    @functools.partial(
        pl.kernel,
        mesh=mesh,
        out_type=jax.ShapeDtypeStruct((batch, dim), jnp.float32),
        scratch_types=[
            pltpu.VMEM((b_per_w,), jnp.int32),
            pltpu.VMEM((b_per_w, dim), jnp.float32),
        ]
        + [pltpu.SemaphoreType.DMA] * (batch // num_workers // _IDX_CHUNK)
        + [pltpu.SemaphoreType.DMA],
    )
    def gather(table_hbm, idx_hbm, out_hbm, idx_v, rows_v, *sems):
        gsems, sem_s = sems[:-1], sems[-1]
        wid = lax.axis_index("s") * info.num_cores + lax.axis_index("c")
        base = wid * b_per_w
        pltpu.sync_copy(idx_hbm.at[pl.ds(base, b_per_w)], idx_v)
        gathers = []
        for j in range(n_chunks):
            off = j * _IDX_CHUNK
            gathers.append(
                pltpu.async_copy(
                    table_hbm.at[idx_v.at[pl.ds(off, _IDX_CHUNK)]],
                    rows_v.at[pl.ds(off, _IDX_CHUNK)],
                    gsems[j],
                )
            )
        scatters = []
        for j in range(n_chunks):
            off = j * _IDX_CHUNK
            gathers[j].wait()
            scatters.append(
                pltpu.async_copy(
                    rows_v.at[pl.ds(off, _IDX_CHUNK)],
                    out_hbm.at[pl.ds(base + off, _IDX_CHUNK)],
                    sem_s,
                )
            )
        for c in scatters:
            c.wait()

    return gather


def _mlp_half(x, w1, b1, w2, b2, w3):
    h = jnp.dot(x, w1, preferred_element_type=jnp.float32)
    h = jnp.maximum(h + b1, 0.0)
    h = jnp.dot(h, w2, preferred_element_type=jnp.float32)
    h = jnp.maximum(h + b2, 0.0)
    z = jnp.dot(h, w3, preferred_element_type=jnp.float32)
    block = z.shape[0]
    row = lax.broadcasted_iota(jnp.int32, (block, 128), 0)
    col = lax.broadcasted_iota(jnp.int32, (block, 128), 1)
    d = jnp.where((row & 127) == col, z, 0.0)
    return jnp.sum(d.reshape(block // 128, 128, 128), axis=1)


def _mlp_body(ea_ref, eb_ref, ec_ref, ed_ref, w1_ref, b1_ref, w2_ref, b2_ref, w3_ref, b3_ref, out_ref):
    w1 = w1_ref[...]
    b1 = b1_ref[...]
    w2 = w2_ref[...]
    b2 = b2_ref[...]
    w3 = w3_ref[...]
    b3 = b3_ref[...]
    for k, ref in enumerate((ea_ref, eb_ref, ec_ref, ed_ref)):
        zk = _mlp_half(ref[...], w1, b1, w2, b2, w3)
        q = zk.shape[0]
        out_ref[k * q : (k + 1) * q, :] = 1.0 / (1.0 + jnp.exp(-(zk + b3)))


def _mlp(embeddings, W1, b1, W2, b2, W3, b3):
    batch = embeddings.shape[0]
    block = 8192
    quarter = block // 4
    grid = (batch // block,)
    out = pl.pallas_call(
        _mlp_body,
        grid=grid,
        in_specs=[
            pl.BlockSpec((quarter, EMBED_DIM), lambda i: (4 * i, 0)),
            pl.BlockSpec((quarter, EMBED_DIM), lambda i: (4 * i + 1, 0)),
            pl.BlockSpec((quarter, EMBED_DIM), lambda i: (4 * i + 2, 0)),
            pl.BlockSpec((quarter, EMBED_DIM), lambda i: (4 * i + 3, 0)),
            pl.BlockSpec((EMBED_DIM, 64), lambda i: (0, 0)),
            pl.BlockSpec((1, 64), lambda i: (0, 0)),
            pl.BlockSpec((64, 32), lambda i: (0, 0)),
            pl.BlockSpec((1, 32), lambda i: (0, 0)),
            pl.BlockSpec((32, 1), lambda i: (0, 0)),
            pl.BlockSpec((1, 1), lambda i: (0, 0)),
        ],
        out_specs=pl.BlockSpec((block // 128, 128), lambda i: (i, 0)),
        out_shape=jax.ShapeDtypeStruct((batch // 128, 128), jnp.float32),
    )(
        embeddings,
        embeddings,
        embeddings,
        embeddings,
        W1,
        b1.reshape(1, 64),
        W2,
        b2.reshape(1, 32),
        W3,
        b3.reshape(1, 1),
    )
    return out.reshape(batch, 1)


def kernel(table, W1, b1, W2, b2, W3, b3, couple_ids):
    idx = couple_ids.astype(jnp.int32)
    embeddings = _make_gather(BATCH, EMBED_DIM)(table, idx)
    x = _mlp(embeddings, W1, b1, W2, b2, W3, b3)
    return (x, embeddings)


# speedup vs baseline: 1.0056x; 1.0056x over previous
import functools

import jax
import jax.numpy as jnp
from jax import lax
from jax.experimental import pallas as pl
from jax.experimental.pallas import tpu as pltpu
from jax.experimental.pallas import tpu_sc as plsc

NUM_COUPLES = 100000
EMBED_DIM = 128
BATCH = 16384

_IDX_CHUNK = 128


@functools.lru_cache(maxsize=None)
def _make_gather(batch: int, dim: int):
    info = plsc.get_sparse_core_info()
    num_workers = info.num_cores * info.num_subcores
    b_per_w = batch // num_workers
    assert batch % (8 * num_workers) == 0
    n_chunks = b_per_w // _IDX_CHUNK
    assert b_per_w % _IDX_CHUNK == 0
    mesh = plsc.VectorSubcoreMesh(core_axis_name="c", subcore_axis_name="s")

    @functools.partial(
        pl.kernel,
        mesh=mesh,
        out_type=jax.ShapeDtypeStruct((batch, dim), jnp.float32),
        scratch_types=[
            pltpu.VMEM((b_per_w,), jnp.int32),
            pltpu.VMEM((b_per_w, dim), jnp.float32),
        ]
        + [pltpu.SemaphoreType.DMA] * (batch // num_workers // _IDX_CHUNK)
        + [pltpu.SemaphoreType.DMA],
    )
    def gather(table_hbm, idx_hbm, out_hbm, idx_v, rows_v, *sems):
        gsems, sem_s = sems[:-1], sems[-1]
        wid = lax.axis_index("s") * info.num_cores + lax.axis_index("c")
        base = wid * b_per_w
        pltpu.sync_copy(idx_hbm.at[pl.ds(base, b_per_w)], idx_v)
        gathers = []
        for j in range(n_chunks):
            off = j * _IDX_CHUNK
            gathers.append(
                pltpu.async_copy(
                    table_hbm.at[idx_v.at[pl.ds(off, _IDX_CHUNK)]],
                    rows_v.at[pl.ds(off, _IDX_CHUNK)],
                    gsems[j],
                )
            )
        scatters = []
        for j in range(n_chunks):
            off = j * _IDX_CHUNK
            gathers[j].wait()
            scatters.append(
                pltpu.async_copy(
                    rows_v.at[pl.ds(off, _IDX_CHUNK)],
                    out_hbm.at[pl.ds(base + off, _IDX_CHUNK)],
                    sem_s,
                )
            )
        for c in scatters:
            c.wait()

    return gather


def _mlp_half(x, w1, b1, w2, b2, w3):
    h = jnp.dot(x, w1, preferred_element_type=jnp.float32)
    h = jnp.maximum(h + b1, 0.0)
    h = jnp.dot(h, w2, preferred_element_type=jnp.float32)
    h = jnp.maximum(h + b2, 0.0)
    z = jnp.dot(h, w3, preferred_element_type=jnp.float32)
    block = z.shape[0]
    row = lax.broadcasted_iota(jnp.int32, (block, 128), 0)
    col = lax.broadcasted_iota(jnp.int32, (block, 128), 1)
    d = jnp.where((row & 127) == col, z, 0.0)
    return jnp.sum(d.reshape(block // 128, 128, 128), axis=1)


def _mlp_body(ea_ref, eb_ref, w1_ref, b1_ref, w2_ref, b2_ref, w3_ref, b3_ref, out_ref):
    w1 = w1_ref[...]
    b1 = b1_ref[...]
    w2 = w2_ref[...]
    b2 = b2_ref[...]
    w3 = w3_ref[...]
    b3 = b3_ref[...]
    for k, ref in enumerate((ea_ref, eb_ref)):
        zk = _mlp_half(ref[...], w1, b1, w2, b2, w3)
        q = zk.shape[0]
        out_ref[k * q : (k + 1) * q, :] = 1.0 / (1.0 + jnp.exp(-(zk + b3)))


def _mlp(embeddings, W1, b1, W2, b2, W3, b3):
    batch = embeddings.shape[0]
    block = 8192
    half = block // 2
    grid = (batch // block,)
    out = pl.pallas_call(
        _mlp_body,
        grid=grid,
        in_specs=[
            pl.BlockSpec((half, EMBED_DIM), lambda i: (2 * i, 0)),
            pl.BlockSpec((half, EMBED_DIM), lambda i: (2 * i + 1, 0)),
            pl.BlockSpec((EMBED_DIM, 64), lambda i: (0, 0)),
            pl.BlockSpec((1, 64), lambda i: (0, 0)),
            pl.BlockSpec((64, 32), lambda i: (0, 0)),
            pl.BlockSpec((1, 32), lambda i: (0, 0)),
            pl.BlockSpec((32, 1), lambda i: (0, 0)),
            pl.BlockSpec((1, 1), lambda i: (0, 0)),
        ],
        out_specs=pl.BlockSpec((block // 128, 128), lambda i: (i, 0)),
        out_shape=jax.ShapeDtypeStruct((batch // 128, 128), jnp.float32),
    )(
        embeddings,
        embeddings,
        W1,
        b1.reshape(1, 64),
        W2,
        b2.reshape(1, 32),
        W3,
        b3.reshape(1, 1),
    )
    return out.reshape(batch, 1)


def kernel(table, W1, b1, W2, b2, W3, b3, couple_ids):
    idx = couple_ids.astype(jnp.int32)
    embeddings = _make_gather(BATCH, EMBED_DIM)(table, idx)
    x = _mlp(embeddings, W1, b1, W2, b2, W3, b3)
    return (x, embeddings)
